# baseline (speedup 1.0000x reference)
import functools

import jax
import jax.numpy as jnp
from jax import lax
from jax.experimental import pallas as pl
from jax.experimental.pallas import tpu as pltpu
from jax.experimental.pallas import tpu_sc as plsc

_N = 10000
_E = 320000
_D = 128
_ALPHA = 0.1

_NC = 2
_NS = 16
_NW = _NC * _NS
_B = 128
_NB = 80
_NBH = _NB // 2
_EPW = _NB * _B
_EPAD = _NW * _EPW
_NPAD = 10240
_RPS = _NPAD // _NS
_DEGW = 16

_mesh = plsc.VectorSubcoreMesh(core_axis_name="c", subcore_axis_name="s")


def _make_sc_scatter(width):
    @functools.partial(
        pl.kernel,
        out_type=jax.ShapeDtypeStruct((_NC, _NPAD, width), jnp.float32),
        mesh=_mesh,
        scratch_types=[
            pltpu.VMEM((_NBH, _B), jnp.int32),
            pltpu.VMEM((_NBH, _B), jnp.int32),
            pltpu.VMEM((_B, width), jnp.float32),
            pltpu.VMEM((_B, width), jnp.float32),
            pltpu.VMEM_SHARED((_NPAD, width), jnp.float32),
            pltpu.SemaphoreType.DMA,
            pltpu.SemaphoreType.DMA,
        ],
    )
    def sc_scatter(g_hbm, src_hbm, dst_hbm, zeros_hbm, out_hbm,
                   src_v, dst_v, rows_a, rows_b, acc, sem_a, sem_b):
        c = lax.axis_index("c")
        s = lax.axis_index("s")
        wid = c * _NS + s
        pltpu.sync_copy(zeros_hbm, acc.at[pl.ds(s * _RPS, _RPS)])
        plsc.subcore_barrier()

        for half in range(2):
            pltpu.sync_copy(src_hbm.at[wid, pl.ds(half * _NBH, _NBH)], src_v)
            pltpu.sync_copy(dst_hbm.at[wid, pl.ds(half * _NBH, _NBH)], dst_v)
            pltpu.async_copy(g_hbm.at[src_v.at[0]], rows_a, sem_a)

            def body(i, carry):
                j0 = 2 * i
                j1 = j0 + 1
                nxt = jnp.minimum(j0 + 2, _NBH - 1)
                pltpu.make_async_copy(
                    g_hbm.at[src_v.at[j0]], rows_a, sem_a).wait()
                pltpu.async_copy(g_hbm.at[src_v.at[j1]], rows_b, sem_b)
                pltpu.sync_copy(rows_a, acc.at[dst_v.at[j0]], add=True)
                pltpu.make_async_copy(
                    g_hbm.at[src_v.at[j1]], rows_b, sem_b).wait()
                pltpu.async_copy(g_hbm.at[src_v.at[nxt]], rows_a, sem_a)
                pltpu.sync_copy(rows_b, acc.at[dst_v.at[j1]], add=True)
                return carry

            lax.fori_loop(0, _NBH // 2, body, 0)
            pltpu.make_async_copy(
                g_hbm.at[src_v.at[_NBH - 1]], rows_a, sem_a).wait()
        plsc.subcore_barrier()
        pltpu.sync_copy(acc.at[pl.ds(s * _RPS, _RPS)],
                        out_hbm.at[c, pl.ds(s * _RPS, _RPS)])

    return sc_scatter


_sc_scatter = _make_sc_scatter(_D)


_RB = 1000
_GRID = _N // _RB


def _mlp_body(x_ref, wi_ref, bi_ref, wh_ref, bh_ref, wo_ref, bo_ref, h_ref):
    h = jnp.dot(x_ref[...], wi_ref[...], preferred_element_type=jnp.float32)
    h = jnp.maximum(h + bi_ref[...], 0.0)
    h = jnp.dot(h, wh_ref[...], preferred_element_type=jnp.float32)
    h = jnp.maximum(h + bh_ref[...], 0.0)
    h = jnp.dot(h, wo_ref[...], preferred_element_type=jnp.float32)
    h_ref[...] = h + bo_ref[...]


def _mlp(x, W_in, b_in, W_h, b_h, W_out, b_out):
    full_w = pl.BlockSpec((_D, _D), lambda i: (0, 0))
    full_b = pl.BlockSpec((1, _D), lambda i: (0, 0))
    rows = pl.BlockSpec((_RB, _D), lambda i: (i, 0))
    return pl.pallas_call(
        _mlp_body,
        grid=(_GRID,),
        in_specs=[rows, full_w, full_b, full_w, full_b, full_w, full_b],
        out_specs=rows,
        out_shape=jax.ShapeDtypeStruct((_N, _D), jnp.float32),
    )(x, W_in, b_in, W_h, b_h, W_out, b_out)


def _prep_body(degp_ref, h0_ref, dinv_ref, g0_ref):
    deg = degp_ref[0, :, 0:1] + degp_ref[1, :, 0:1] + 1.0
    dinv = lax.rsqrt(deg)
    dinv_ref[...] = dinv
    g0_ref[...] = dinv * h0_ref[...]


def _prep(degp, h0):
    return pl.pallas_call(
        _prep_body,
        grid=(_GRID,),
        in_specs=[
            pl.BlockSpec((_NC, _RB, _D), lambda i: (0, i, 0)),
            pl.BlockSpec((_RB, _D), lambda i: (i, 0)),
        ],
        out_specs=[
            pl.BlockSpec((_RB, 1), lambda i: (i, 0)),
            pl.BlockSpec((_RB, _D), lambda i: (i, 0)),
        ],
        out_shape=[
            jax.ShapeDtypeStruct((_N, 1), jnp.float32),
            jax.ShapeDtypeStruct((_N, _D), jnp.float32),
        ],
    )(degp, h0)


def _combine_body_g(p_ref, hc_ref, h0_ref, dinv_ref, hn_ref, gn_ref):
    dinv = dinv_ref[...]
    agg = dinv * (p_ref[0] + p_ref[1]) + dinv * dinv * hc_ref[...]
    hn = (1.0 - _ALPHA) * agg + _ALPHA * h0_ref[...]
    hn_ref[...] = hn
    gn_ref[...] = dinv * hn


def _combine_body(p_ref, hc_ref, h0_ref, dinv_ref, hn_ref):
    dinv = dinv_ref[...]
    agg = dinv * (p_ref[0] + p_ref[1]) + dinv * dinv * hc_ref[...]
    hn_ref[...] = (1.0 - _ALPHA) * agg + _ALPHA * h0_ref[...]


def _combine(p, h_cur, h0, dinv, with_g):
    rows = pl.BlockSpec((_RB, _D), lambda i: (i, 0))
    in_specs = [
        pl.BlockSpec((_NC, _RB, _D), lambda i: (0, i, 0)),
        rows, rows,
        pl.BlockSpec((_RB, 1), lambda i: (i, 0)),
    ]
    if with_g:
        return pl.pallas_call(
            _combine_body_g,
            grid=(_GRID,),
            in_specs=in_specs,
            out_specs=[rows, rows],
            out_shape=[jax.ShapeDtypeStruct((_N, _D), jnp.float32)] * 2,
        )(p, h_cur, h0, dinv)
    return pl.pallas_call(
        _combine_body,
        grid=(_GRID,),
        in_specs=in_specs,
        out_specs=rows,
        out_shape=jax.ShapeDtypeStruct((_N, _D), jnp.float32),
    )(p, h_cur, h0, dinv)


def kernel(x, edge_index, W_in, b_in, W_h, b_h, W_out, b_out):
    ei = edge_index.astype(jnp.int32)
    npad = _EPAD - _E
    src = jnp.concatenate(
        [ei[0], jnp.zeros((npad,), jnp.int32)]).reshape(_NW, _NB, _B)
    dst = jnp.concatenate(
        [ei[1], _N + (jnp.arange(npad, dtype=jnp.int32) % (_NPAD - _N))]
    ).reshape(_NW, _NB, _B)
    zeros_d = jnp.zeros((_RPS, _D), jnp.float32)
    ones_nd = jnp.ones((_N, _D), jnp.float32)

    h0 = _mlp(x, W_in, b_in.reshape(1, _D), W_h, b_h.reshape(1, _D),
              W_out, b_out.reshape(1, _D))
    degp = _sc_scatter(ones_nd, src, dst, zeros_d)
    dinv, g0 = _prep(degp, h0)
    p1 = _sc_scatter(g0, src, dst, zeros_d)
    h1, g1 = _combine(p1, h0, h0, dinv, with_g=True)
    p2 = _sc_scatter(g1, src, dst, zeros_d)
    return _combine(p2, h1, h0, dinv, with_g=False)


# speedup vs baseline: 1.0629x; 1.0629x over previous
import functools

import jax
import jax.numpy as jnp
from jax import lax
from jax.experimental import pallas as pl
from jax.experimental.pallas import tpu as pltpu
from jax.experimental.pallas import tpu_sc as plsc

_N = 10000
_E = 320000
_D = 128
_ALPHA = 0.1

_NC = 2
_NS = 16
_NW = _NC * _NS
_B = 128
_NB = 80
_NBH = _NB // 2
_EPW = _NB * _B
_EPAD = _NW * _EPW
_NPAD = 10240
_RPS = _NPAD // _NS
_DEGW = 16

_mesh = plsc.VectorSubcoreMesh(core_axis_name="c", subcore_axis_name="s")


def _make_sc_scatter(width):
    @functools.partial(
        pl.kernel,
        out_type=jax.ShapeDtypeStruct((_NC, _NPAD, width), jnp.float32),
        mesh=_mesh,
        scratch_types=[
            pltpu.VMEM((_NBH, _B), jnp.int32),
            pltpu.VMEM((_NBH, _B), jnp.int32),
            pltpu.VMEM((_B, width), jnp.float32),
            pltpu.VMEM((_B, width), jnp.float32),
            pltpu.VMEM_SHARED((_NPAD, width), jnp.float32),
            pltpu.SemaphoreType.DMA,
            pltpu.SemaphoreType.DMA,
            pltpu.SemaphoreType.DMA,
        ],
    )
    def sc_scatter(g_hbm, src_hbm, dst_hbm, zeros_hbm, out_hbm,
                   src_v, dst_v, rows_a, rows_b, acc, sem_g, sem_a, sem_b):
        c = lax.axis_index("c")
        s = lax.axis_index("s")
        wid = c * _NS + s
        pltpu.sync_copy(zeros_hbm, acc.at[pl.ds(s * _RPS, _RPS)])
        plsc.subcore_barrier()

        def drain(buf, sem):
            pltpu.make_async_copy(g_hbm.at[pl.ds(0, _B)], buf, sem).wait()

        for half in range(2):
            pltpu.sync_copy(src_hbm.at[wid, pl.ds(half * _NBH, _NBH)], src_v)
            pltpu.sync_copy(dst_hbm.at[wid, pl.ds(half * _NBH, _NBH)], dst_v)
            pltpu.async_copy(g_hbm.at[src_v.at[0]], rows_a, sem_g).wait()
            pltpu.async_copy(rows_a, acc.at[dst_v.at[0]], sem_a, add=True)
            pltpu.async_copy(g_hbm.at[src_v.at[1]], rows_b, sem_g).wait()
            pltpu.async_copy(rows_b, acc.at[dst_v.at[1]], sem_b, add=True)

            def body(i, carry):
                j0 = 2 * i
                j1 = j0 + 1
                drain(rows_a, sem_a)
                pltpu.async_copy(g_hbm.at[src_v.at[j0]], rows_a, sem_g).wait()
                pltpu.async_copy(rows_a, acc.at[dst_v.at[j0]], sem_a, add=True)
                drain(rows_b, sem_b)
                pltpu.async_copy(g_hbm.at[src_v.at[j1]], rows_b, sem_g).wait()
                pltpu.async_copy(rows_b, acc.at[dst_v.at[j1]], sem_b, add=True)
                return carry

            lax.fori_loop(1, _NBH // 2, body, 0)
            drain(rows_a, sem_a)
            drain(rows_b, sem_b)
        plsc.subcore_barrier()
        pltpu.sync_copy(acc.at[pl.ds(s * _RPS, _RPS)],
                        out_hbm.at[c, pl.ds(s * _RPS, _RPS)])

    return sc_scatter


_sc_scatter = _make_sc_scatter(_D)


_RB = 1000
_GRID = _N // _RB


def _mlp_body(x_ref, wi_ref, bi_ref, wh_ref, bh_ref, wo_ref, bo_ref, h_ref):
    h = jnp.dot(x_ref[...], wi_ref[...], preferred_element_type=jnp.float32)
    h = jnp.maximum(h + bi_ref[...], 0.0)
    h = jnp.dot(h, wh_ref[...], preferred_element_type=jnp.float32)
    h = jnp.maximum(h + bh_ref[...], 0.0)
    h = jnp.dot(h, wo_ref[...], preferred_element_type=jnp.float32)
    h_ref[...] = h + bo_ref[...]


def _mlp(x, W_in, b_in, W_h, b_h, W_out, b_out):
    full_w = pl.BlockSpec((_D, _D), lambda i: (0, 0))
    full_b = pl.BlockSpec((1, _D), lambda i: (0, 0))
    rows = pl.BlockSpec((_RB, _D), lambda i: (i, 0))
    return pl.pallas_call(
        _mlp_body,
        grid=(_GRID,),
        in_specs=[rows, full_w, full_b, full_w, full_b, full_w, full_b],
        out_specs=rows,
        out_shape=jax.ShapeDtypeStruct((_N, _D), jnp.float32),
    )(x, W_in, b_in, W_h, b_h, W_out, b_out)


def _prep_body(degp_ref, h0_ref, dinv_ref, g0_ref):
    deg = degp_ref[0, :, 0:1] + degp_ref[1, :, 0:1] + 1.0
    dinv = lax.rsqrt(deg)
    dinv_ref[...] = dinv
    g0_ref[...] = dinv * h0_ref[...]


def _prep(degp, h0):
    return pl.pallas_call(
        _prep_body,
        grid=(_GRID,),
        in_specs=[
            pl.BlockSpec((_NC, _RB, _D), lambda i: (0, i, 0)),
            pl.BlockSpec((_RB, _D), lambda i: (i, 0)),
        ],
        out_specs=[
            pl.BlockSpec((_RB, 1), lambda i: (i, 0)),
            pl.BlockSpec((_RB, _D), lambda i: (i, 0)),
        ],
        out_shape=[
            jax.ShapeDtypeStruct((_N, 1), jnp.float32),
            jax.ShapeDtypeStruct((_N, _D), jnp.float32),
        ],
    )(degp, h0)


def _combine_body_g(p_ref, hc_ref, h0_ref, dinv_ref, hn_ref, gn_ref):
    dinv = dinv_ref[...]
    agg = dinv * (p_ref[0] + p_ref[1]) + dinv * dinv * hc_ref[...]
    hn = (1.0 - _ALPHA) * agg + _ALPHA * h0_ref[...]
    hn_ref[...] = hn
    gn_ref[...] = dinv * hn


def _combine_body(p_ref, hc_ref, h0_ref, dinv_ref, hn_ref):
    dinv = dinv_ref[...]
    agg = dinv * (p_ref[0] + p_ref[1]) + dinv * dinv * hc_ref[...]
    hn_ref[...] = (1.0 - _ALPHA) * agg + _ALPHA * h0_ref[...]


def _combine(p, h_cur, h0, dinv, with_g):
    rows = pl.BlockSpec((_RB, _D), lambda i: (i, 0))
    in_specs = [
        pl.BlockSpec((_NC, _RB, _D), lambda i: (0, i, 0)),
        rows, rows,
        pl.BlockSpec((_RB, 1), lambda i: (i, 0)),
    ]
    if with_g:
        return pl.pallas_call(
            _combine_body_g,
            grid=(_GRID,),
            in_specs=in_specs,
            out_specs=[rows, rows],
            out_shape=[jax.ShapeDtypeStruct((_N, _D), jnp.float32)] * 2,
        )(p, h_cur, h0, dinv)
    return pl.pallas_call(
        _combine_body,
        grid=(_GRID,),
        in_specs=in_specs,
        out_specs=rows,
        out_shape=jax.ShapeDtypeStruct((_N, _D), jnp.float32),
    )(p, h_cur, h0, dinv)


def kernel(x, edge_index, W_in, b_in, W_h, b_h, W_out, b_out):
    ei = edge_index.astype(jnp.int32)
    npad = _EPAD - _E
    src = jnp.concatenate(
        [ei[0], jnp.zeros((npad,), jnp.int32)]).reshape(_NW, _NB, _B)
    dst = jnp.concatenate(
        [ei[1], _N + (jnp.arange(npad, dtype=jnp.int32) % (_NPAD - _N))]
    ).reshape(_NW, _NB, _B)
    zeros_d = jnp.zeros((_RPS, _D), jnp.float32)
    ones_nd = jnp.ones((_N, _D), jnp.float32)

    h0 = _mlp(x, W_in, b_in.reshape(1, _D), W_h, b_h.reshape(1, _D),
              W_out, b_out.reshape(1, _D))
    degp = _sc_scatter(ones_nd, src, dst, zeros_d)
    dinv, g0 = _prep(degp, h0)
    p1 = _sc_scatter(g0, src, dst, zeros_d)
    h1, g1 = _combine(p1, h0, h0, dinv, with_g=True)
    p2 = _sc_scatter(g1, src, dst, zeros_d)
    return _combine(p2, h1, h0, dinv, with_g=False)


# speedup vs baseline: 1.0924x; 1.0277x over previous
import functools

import jax
import jax.numpy as jnp
from jax import lax
from jax.experimental import pallas as pl
from jax.experimental.pallas import tpu as pltpu
from jax.experimental.pallas import tpu_sc as plsc

_N = 10000
_E = 320000
_D = 128
_ALPHA = 0.1

_NC = 2
_NS = 16
_NW = _NC * _NS
_B = 80
_NB = 128
_NBH = _NB // 2
_EPW = _NB * _B
_EPAD = _NW * _EPW
_NPAD = 10240
_RPS = _NPAD // _NS
_DEGW = 16

_mesh = plsc.VectorSubcoreMesh(core_axis_name="c", subcore_axis_name="s")


def _make_sc_scatter(width):
    @functools.partial(
        pl.kernel,
        out_type=jax.ShapeDtypeStruct((_NC, _NPAD, width), jnp.float32),
        mesh=_mesh,
        scratch_types=[
            pltpu.VMEM((_NBH, _B), jnp.int32),
            pltpu.VMEM((_NBH, _B), jnp.int32),
            pltpu.VMEM((_B, width), jnp.float32),
            pltpu.VMEM((_B, width), jnp.float32),
            pltpu.VMEM_SHARED((_NPAD, width), jnp.float32),
            pltpu.SemaphoreType.DMA,
            pltpu.SemaphoreType.DMA,
            pltpu.SemaphoreType.DMA,
        ],
    )
    def sc_scatter(g_hbm, src_hbm, dst_hbm, zeros_hbm, out_hbm,
                   src_v, dst_v, rows_a, rows_b, acc, sem_g, sem_a, sem_b):
        c = lax.axis_index("c")
        s = lax.axis_index("s")
        wid = c * _NS + s
        pltpu.sync_copy(zeros_hbm, acc.at[pl.ds(s * _RPS, _RPS)])
        plsc.subcore_barrier()

        def drain(buf, sem):
            pltpu.make_async_copy(g_hbm.at[pl.ds(0, _B)], buf, sem).wait()

        for half in range(2):
            pltpu.sync_copy(src_hbm.at[wid, pl.ds(half * _NBH, _NBH)], src_v)
            pltpu.sync_copy(dst_hbm.at[wid, pl.ds(half * _NBH, _NBH)], dst_v)
            pltpu.async_copy(g_hbm.at[src_v.at[0]], rows_a, sem_g).wait()
            pltpu.async_copy(rows_a, acc.at[dst_v.at[0]], sem_a, add=True)
            pltpu.async_copy(g_hbm.at[src_v.at[1]], rows_b, sem_g).wait()
            pltpu.async_copy(rows_b, acc.at[dst_v.at[1]], sem_b, add=True)

            def body(i, carry):
                j0 = 2 * i
                j1 = j0 + 1
                drain(rows_a, sem_a)
                pltpu.async_copy(g_hbm.at[src_v.at[j0]], rows_a, sem_g).wait()
                pltpu.async_copy(rows_a, acc.at[dst_v.at[j0]], sem_a, add=True)
                drain(rows_b, sem_b)
                pltpu.async_copy(g_hbm.at[src_v.at[j1]], rows_b, sem_g).wait()
                pltpu.async_copy(rows_b, acc.at[dst_v.at[j1]], sem_b, add=True)
                return carry

            lax.fori_loop(1, _NBH // 2, body, 0)
            drain(rows_a, sem_a)
            drain(rows_b, sem_b)
        plsc.subcore_barrier()
        pltpu.sync_copy(acc.at[pl.ds(s * _RPS, _RPS)],
                        out_hbm.at[c, pl.ds(s * _RPS, _RPS)])

    return sc_scatter


_sc_scatter = _make_sc_scatter(_D)


_RB = 1000
_GRID = _N // _RB


def _mlp_body(x_ref, wi_ref, bi_ref, wh_ref, bh_ref, wo_ref, bo_ref, h_ref):
    h = jnp.dot(x_ref[...], wi_ref[...], preferred_element_type=jnp.float32)
    h = jnp.maximum(h + bi_ref[...], 0.0)
    h = jnp.dot(h, wh_ref[...], preferred_element_type=jnp.float32)
    h = jnp.maximum(h + bh_ref[...], 0.0)
    h = jnp.dot(h, wo_ref[...], preferred_element_type=jnp.float32)
    h_ref[...] = h + bo_ref[...]


def _mlp(x, W_in, b_in, W_h, b_h, W_out, b_out):
    full_w = pl.BlockSpec((_D, _D), lambda i: (0, 0))
    full_b = pl.BlockSpec((1, _D), lambda i: (0, 0))
    rows = pl.BlockSpec((_RB, _D), lambda i: (i, 0))
    return pl.pallas_call(
        _mlp_body,
        grid=(_GRID,),
        in_specs=[rows, full_w, full_b, full_w, full_b, full_w, full_b],
        out_specs=rows,
        out_shape=jax.ShapeDtypeStruct((_N, _D), jnp.float32),
    )(x, W_in, b_in, W_h, b_h, W_out, b_out)


def _prep_body(degp_ref, h0_ref, dinv_ref, g0_ref):
    deg = degp_ref[0, :, 0:1] + degp_ref[1, :, 0:1] + 1.0
    dinv = lax.rsqrt(deg)
    dinv_ref[...] = dinv
    g0_ref[...] = dinv * h0_ref[...]


def _prep(degp, h0):
    return pl.pallas_call(
        _prep_body,
        grid=(_GRID,),
        in_specs=[
            pl.BlockSpec((_NC, _RB, _D), lambda i: (0, i, 0)),
            pl.BlockSpec((_RB, _D), lambda i: (i, 0)),
        ],
        out_specs=[
            pl.BlockSpec((_RB, 1), lambda i: (i, 0)),
            pl.BlockSpec((_RB, _D), lambda i: (i, 0)),
        ],
        out_shape=[
            jax.ShapeDtypeStruct((_N, 1), jnp.float32),
            jax.ShapeDtypeStruct((_N, _D), jnp.float32),
        ],
    )(degp, h0)


def _combine_body_g(p_ref, hc_ref, h0_ref, dinv_ref, hn_ref, gn_ref):
    dinv = dinv_ref[...]
    agg = dinv * (p_ref[0] + p_ref[1]) + dinv * dinv * hc_ref[...]
    hn = (1.0 - _ALPHA) * agg + _ALPHA * h0_ref[...]
    hn_ref[...] = hn
    gn_ref[...] = dinv * hn


def _combine_body(p_ref, hc_ref, h0_ref, dinv_ref, hn_ref):
    dinv = dinv_ref[...]
    agg = dinv * (p_ref[0] + p_ref[1]) + dinv * dinv * hc_ref[...]
    hn_ref[...] = (1.0 - _ALPHA) * agg + _ALPHA * h0_ref[...]


def _combine(p, h_cur, h0, dinv, with_g):
    rows = pl.BlockSpec((_RB, _D), lambda i: (i, 0))
    in_specs = [
        pl.BlockSpec((_NC, _RB, _D), lambda i: (0, i, 0)),
        rows, rows,
        pl.BlockSpec((_RB, 1), lambda i: (i, 0)),
    ]
    if with_g:
        return pl.pallas_call(
            _combine_body_g,
            grid=(_GRID,),
            in_specs=in_specs,
            out_specs=[rows, rows],
            out_shape=[jax.ShapeDtypeStruct((_N, _D), jnp.float32)] * 2,
        )(p, h_cur, h0, dinv)
    return pl.pallas_call(
        _combine_body,
        grid=(_GRID,),
        in_specs=in_specs,
        out_specs=rows,
        out_shape=jax.ShapeDtypeStruct((_N, _D), jnp.float32),
    )(p, h_cur, h0, dinv)


def kernel(x, edge_index, W_in, b_in, W_h, b_h, W_out, b_out):
    ei = edge_index.astype(jnp.int32)
    npad = _EPAD - _E
    src = jnp.concatenate(
        [ei[0], jnp.zeros((npad,), jnp.int32)]).reshape(_NW, _NB, _B)
    dst = jnp.concatenate(
        [ei[1], _N + (jnp.arange(npad, dtype=jnp.int32) % (_NPAD - _N))]
    ).reshape(_NW, _NB, _B)
    zeros_d = jnp.zeros((_RPS, _D), jnp.float32)
    ones_nd = jnp.ones((_N, _D), jnp.float32)

    h0 = _mlp(x, W_in, b_in.reshape(1, _D), W_h, b_h.reshape(1, _D),
              W_out, b_out.reshape(1, _D))
    degp = _sc_scatter(ones_nd, src, dst, zeros_d)
    dinv, g0 = _prep(degp, h0)
    p1 = _sc_scatter(g0, src, dst, zeros_d)
    h1, g1 = _combine(p1, h0, h0, dinv, with_g=True)
    p2 = _sc_scatter(g1, src, dst, zeros_d)
    return _combine(p2, h1, h0, dinv, with_g=False)


# speedup vs baseline: 1.1646x; 1.0662x over previous
import functools

import jax
import jax.numpy as jnp
from jax import lax
from jax.experimental import pallas as pl
from jax.experimental.pallas import tpu as pltpu
from jax.experimental.pallas import tpu_sc as plsc

_N = 10000
_E = 320000
_D = 128
_ALPHA = 0.1

_NC = 2
_NS = 16
_NW = _NC * _NS
_B = 80
_NB = 128
_NBH = _NB // 2
_EPW = _NB * _B
_EPAD = _NW * _EPW
_NPAD = 10240
_RPS = _NPAD // _NS
_DEGW = 16

_mesh = plsc.VectorSubcoreMesh(core_axis_name="c", subcore_axis_name="s")


def _make_sc_scatter(width):
    @functools.partial(
        pl.kernel,
        out_type=jax.ShapeDtypeStruct((_NC, _NPAD, width), jnp.float32),
        mesh=_mesh,
        scratch_types=[
            pltpu.VMEM((_NBH, _B), jnp.int32),
            pltpu.VMEM((_NBH, _B), jnp.int32),
            pltpu.VMEM((_B, width), jnp.float32),
            pltpu.VMEM((_B, width), jnp.float32),
            pltpu.VMEM_SHARED((_NPAD, width), jnp.float32),
            pltpu.SemaphoreType.DMA,
            pltpu.SemaphoreType.DMA,
            pltpu.SemaphoreType.DMA,
        ],
    )
    def sc_scatter(g_hbm, src_hbm, dst_hbm, zeros_hbm, out_hbm,
                   src_v, dst_v, rows_a, rows_b, acc, sem_g, sem_a, sem_b):
        c = lax.axis_index("c")
        s = lax.axis_index("s")
        wid = c * _NS + s
        pltpu.sync_copy(zeros_hbm, acc.at[pl.ds(s * _RPS, _RPS)])
        plsc.subcore_barrier()

        def drain(buf, sem):
            pltpu.make_async_copy(g_hbm.at[pl.ds(0, _B)], buf, sem).wait()

        for half in range(2):
            pltpu.sync_copy(src_hbm.at[wid, pl.ds(half * _NBH, _NBH)], src_v)
            pltpu.sync_copy(dst_hbm.at[wid, pl.ds(half * _NBH, _NBH)], dst_v)
            pltpu.async_copy(g_hbm.at[src_v.at[0]], rows_a, sem_g).wait()
            pltpu.async_copy(rows_a, acc.at[dst_v.at[0]], sem_a, add=True)
            pltpu.async_copy(g_hbm.at[src_v.at[1]], rows_b, sem_g).wait()
            pltpu.async_copy(rows_b, acc.at[dst_v.at[1]], sem_b, add=True)

            def body(i, carry):
                j0 = 2 * i
                j1 = j0 + 1
                drain(rows_a, sem_a)
                pltpu.async_copy(g_hbm.at[src_v.at[j0]], rows_a, sem_g).wait()
                pltpu.async_copy(rows_a, acc.at[dst_v.at[j0]], sem_a, add=True)
                drain(rows_b, sem_b)
                pltpu.async_copy(g_hbm.at[src_v.at[j1]], rows_b, sem_g).wait()
                pltpu.async_copy(rows_b, acc.at[dst_v.at[j1]], sem_b, add=True)
                return carry

            lax.fori_loop(1, _NBH // 2, body, 0)
            drain(rows_a, sem_a)
            drain(rows_b, sem_b)
        plsc.subcore_barrier()
        pltpu.sync_copy(acc.at[pl.ds(s * _RPS, _RPS)],
                        out_hbm.at[c, pl.ds(s * _RPS, _RPS)])

    return sc_scatter


_sc_scatter = _make_sc_scatter(_D)


_RB = 1000
_GRID = _N // _RB


def _mlp_body(x_ref, wi_ref, bi_ref, wh_ref, bh_ref, wo_ref, bo_ref, h_ref):
    h = jnp.dot(x_ref[...], wi_ref[...], preferred_element_type=jnp.float32)
    h = jnp.maximum(h + bi_ref[...], 0.0)
    h = jnp.dot(h, wh_ref[...], preferred_element_type=jnp.float32)
    h = jnp.maximum(h + bh_ref[...], 0.0)
    h = jnp.dot(h, wo_ref[...], preferred_element_type=jnp.float32)
    h_ref[...] = h + bo_ref[...]


def _mlp(x, W_in, b_in, W_h, b_h, W_out, b_out):
    full_w = pl.BlockSpec((_D, _D), lambda i: (0, 0))
    full_b = pl.BlockSpec((1, _D), lambda i: (0, 0))
    rows = pl.BlockSpec((_RB, _D), lambda i: (i, 0))
    return pl.pallas_call(
        _mlp_body,
        grid=(_GRID,),
        in_specs=[rows, full_w, full_b, full_w, full_b, full_w, full_b],
        out_specs=rows,
        out_shape=jax.ShapeDtypeStruct((_N, _D), jnp.float32),
    )(x, W_in, b_in, W_h, b_h, W_out, b_out)


def _prep_body(degp_ref, h0_ref, dinv_ref, g0_ref):
    deg = degp_ref[0, :, 0:1] + degp_ref[1, :, 0:1] + 1.0
    dinv = lax.rsqrt(deg)
    dinv_ref[...] = dinv
    g0_ref[...] = dinv * h0_ref[...]


def _prep(degp, h0):
    return pl.pallas_call(
        _prep_body,
        grid=(_GRID,),
        in_specs=[
            pl.BlockSpec((_NC, _RB, _D), lambda i: (0, i, 0)),
            pl.BlockSpec((_RB, _D), lambda i: (i, 0)),
        ],
        out_specs=[
            pl.BlockSpec((_RB, 1), lambda i: (i, 0)),
            pl.BlockSpec((_RB, _D), lambda i: (i, 0)),
        ],
        out_shape=[
            jax.ShapeDtypeStruct((_N, 1), jnp.float32),
            jax.ShapeDtypeStruct((_N, _D), jnp.float32),
        ],
    )(degp, h0)


def _combine_body_g(p_ref, hc_ref, h0_ref, dinv_ref, hn_ref, gn_ref):
    dinv = dinv_ref[...]
    agg = dinv * (p_ref[0] + p_ref[1]) + dinv * dinv * hc_ref[...]
    hn = (1.0 - _ALPHA) * agg + _ALPHA * h0_ref[...]
    hn_ref[...] = hn
    gn_ref[...] = dinv * hn


def _combine_body(p_ref, hc_ref, h0_ref, dinv_ref, hn_ref):
    dinv = dinv_ref[...]
    agg = dinv * (p_ref[0] + p_ref[1]) + dinv * dinv * hc_ref[...]
    hn_ref[...] = (1.0 - _ALPHA) * agg + _ALPHA * h0_ref[...]


def _combine(p, h_cur, h0, dinv, with_g):
    rows = pl.BlockSpec((_RB, _D), lambda i: (i, 0))
    in_specs = [
        pl.BlockSpec((_NC, _RB, _D), lambda i: (0, i, 0)),
        rows, rows,
        pl.BlockSpec((_RB, 1), lambda i: (i, 0)),
    ]
    if with_g:
        return pl.pallas_call(
            _combine_body_g,
            grid=(_GRID,),
            in_specs=in_specs,
            out_specs=[rows, rows],
            out_shape=[jax.ShapeDtypeStruct((_N, _D), jnp.float32)] * 2,
        )(p, h_cur, h0, dinv)
    return pl.pallas_call(
        _combine_body,
        grid=(_GRID,),
        in_specs=in_specs,
        out_specs=rows,
        out_shape=jax.ShapeDtypeStruct((_N, _D), jnp.float32),
    )(p, h_cur, h0, dinv)


def kernel(x, edge_index, W_in, b_in, W_h, b_h, W_out, b_out):
    ei = edge_index.astype(jnp.int32)
    ppw = _EPW - _E // _NW
    pad_src = jnp.zeros((_NW, ppw), jnp.int32)
    pad_dst = jnp.broadcast_to(
        _N + jnp.arange(ppw, dtype=jnp.int32), (_NW, ppw))
    src = jnp.concatenate(
        [ei[0].reshape(_NW, -1), pad_src], axis=1).reshape(_NW, _NB, _B)
    dst = jnp.concatenate(
        [ei[1].reshape(_NW, -1), pad_dst], axis=1).reshape(_NW, _NB, _B)
    zeros_d = jnp.zeros((_RPS, _D), jnp.float32)
    ones_nd = jnp.ones((_N, _D), jnp.float32)

    h0 = _mlp(x, W_in, b_in.reshape(1, _D), W_h, b_h.reshape(1, _D),
              W_out, b_out.reshape(1, _D))
    degp = _sc_scatter(ones_nd, src, dst, zeros_d)
    dinv, g0 = _prep(degp, h0)
    p1 = _sc_scatter(g0, src, dst, zeros_d)
    h1, g1 = _combine(p1, h0, h0, dinv, with_g=True)
    p2 = _sc_scatter(g1, src, dst, zeros_d)
    return _combine(p2, h1, h0, dinv, with_g=False)


# speedup vs baseline: 2.9062x; 2.4954x over previous
import functools

import jax
import jax.numpy as jnp
from jax import lax
from jax.experimental import pallas as pl
from jax.experimental.pallas import tpu as pltpu
from jax.experimental.pallas import tpu_sc as plsc

_N = 10000
_E = 320000
_D = 128
_ALPHA = 0.1

_NC = 2
_NS = 16
_NW = _NC * _NS
_EPW = _E // _NW
_B = 80
_NB = _EPW // _B
_NPAD = 10240
_RPS = _NPAD // _NS

_mesh = plsc.VectorSubcoreMesh(core_axis_name="c", subcore_axis_name="s")


@functools.partial(
    pl.kernel,
    out_type=jax.ShapeDtypeStruct((_NC, _NPAD, _D), jnp.float32),
    mesh=_mesh,
    scratch_types=[
        pltpu.VMEM((_NB, _B), jnp.int32),
        pltpu.VMEM((_NB, _B), jnp.int32),
        pltpu.VMEM((_B, _D), jnp.float32),
        pltpu.VMEM_SHARED((_NPAD, _D), jnp.float32),
        pltpu.SemaphoreType.DMA,
    ],
)
def _sc_scatter(g_hbm, src_hbm, dst_hbm, zeros_hbm, out_hbm,
                src_v, dst_v, rows_v, acc, sem):
    c = lax.axis_index("c")
    s = lax.axis_index("s")
    wid = c * _NS + s
    pltpu.sync_copy(src_hbm.at[wid], src_v)
    pltpu.sync_copy(dst_hbm.at[wid], dst_v)
    pltpu.sync_copy(zeros_hbm, acc.at[pl.ds(s * _RPS, _RPS)])
    plsc.subcore_barrier()

    def body(j, carry):
        pltpu.async_copy(g_hbm.at[src_v.at[j]], rows_v, sem).wait()
        pltpu.sync_copy(rows_v, acc.at[dst_v.at[j]], add=True)
        return carry

    lax.fori_loop(0, _NB, body, 0)
    plsc.subcore_barrier()
    pltpu.sync_copy(acc.at[pl.ds(s * _RPS, _RPS)],
                    out_hbm.at[c, pl.ds(s * _RPS, _RPS)])


@functools.partial(
    pl.kernel,
    out_type=jax.ShapeDtypeStruct((_NC, _NPAD, _D), jnp.float32),
    mesh=_mesh,
    scratch_types=[
        pltpu.VMEM((_NB, _B), jnp.int32),
        pltpu.VMEM((_B, _D), jnp.float32),
        pltpu.VMEM_SHARED((_NPAD, _D), jnp.float32),
    ],
)
def _sc_degree(ones_hbm, dst_hbm, zeros_hbm, out_hbm, dst_v, ones_v, acc):
    c = lax.axis_index("c")
    s = lax.axis_index("s")
    wid = c * _NS + s
    pltpu.sync_copy(dst_hbm.at[wid], dst_v)
    pltpu.sync_copy(ones_hbm, ones_v)
    pltpu.sync_copy(zeros_hbm, acc.at[pl.ds(s * _RPS, _RPS)])
    plsc.subcore_barrier()

    def body(j, carry):
        pltpu.sync_copy(ones_v, acc.at[dst_v.at[j]], add=True)
        return carry

    lax.fori_loop(0, _NB, body, 0)
    plsc.subcore_barrier()
    pltpu.sync_copy(acc.at[pl.ds(s * _RPS, _RPS)],
                    out_hbm.at[c, pl.ds(s * _RPS, _RPS)])


_RB = 1000
_GRID = _N // _RB


def _mlp_body(x_ref, wi_ref, bi_ref, wh_ref, bh_ref, wo_ref, bo_ref, h_ref):
    h = jnp.dot(x_ref[...], wi_ref[...], preferred_element_type=jnp.float32)
    h = jnp.maximum(h + bi_ref[...], 0.0)
    h = jnp.dot(h, wh_ref[...], preferred_element_type=jnp.float32)
    h = jnp.maximum(h + bh_ref[...], 0.0)
    h = jnp.dot(h, wo_ref[...], preferred_element_type=jnp.float32)
    h_ref[...] = h + bo_ref[...]


def _mlp(x, W_in, b_in, W_h, b_h, W_out, b_out):
    full_w = pl.BlockSpec((_D, _D), lambda i: (0, 0))
    full_b = pl.BlockSpec((1, _D), lambda i: (0, 0))
    rows = pl.BlockSpec((_RB, _D), lambda i: (i, 0))
    return pl.pallas_call(
        _mlp_body,
        grid=(_GRID,),
        in_specs=[rows, full_w, full_b, full_w, full_b, full_w, full_b],
        out_specs=rows,
        out_shape=jax.ShapeDtypeStruct((_N, _D), jnp.float32),
    )(x, W_in, b_in, W_h, b_h, W_out, b_out)


def _prep_body(degp_ref, h0_ref, dinv_ref, g0_ref):
    deg = degp_ref[0, :, 0:1] + degp_ref[1, :, 0:1] + 1.0
    dinv = lax.rsqrt(deg)
    dinv_ref[...] = dinv
    g0_ref[...] = dinv * h0_ref[...]


def _prep(degp, h0):
    return pl.pallas_call(
        _prep_body,
        grid=(_GRID,),
        in_specs=[
            pl.BlockSpec((_NC, _RB, _D), lambda i: (0, i, 0)),
            pl.BlockSpec((_RB, _D), lambda i: (i, 0)),
        ],
        out_specs=[
            pl.BlockSpec((_RB, 1), lambda i: (i, 0)),
            pl.BlockSpec((_RB, _D), lambda i: (i, 0)),
        ],
        out_shape=[
            jax.ShapeDtypeStruct((_N, 1), jnp.float32),
            jax.ShapeDtypeStruct((_N, _D), jnp.float32),
        ],
    )(degp, h0)


def _combine_body_g(p_ref, hc_ref, h0_ref, dinv_ref, hn_ref, gn_ref):
    dinv = dinv_ref[...]
    agg = dinv * (p_ref[0] + p_ref[1]) + dinv * dinv * hc_ref[...]
    hn = (1.0 - _ALPHA) * agg + _ALPHA * h0_ref[...]
    hn_ref[...] = hn
    gn_ref[...] = dinv * hn


def _combine_body(p_ref, hc_ref, h0_ref, dinv_ref, hn_ref):
    dinv = dinv_ref[...]
    agg = dinv * (p_ref[0] + p_ref[1]) + dinv * dinv * hc_ref[...]
    hn_ref[...] = (1.0 - _ALPHA) * agg + _ALPHA * h0_ref[...]


def _combine(p, h_cur, h0, dinv, with_g):
    rows = pl.BlockSpec((_RB, _D), lambda i: (i, 0))
    in_specs = [
        pl.BlockSpec((_NC, _RB, _D), lambda i: (0, i, 0)),
        rows, rows,
        pl.BlockSpec((_RB, 1), lambda i: (i, 0)),
    ]
    if with_g:
        return pl.pallas_call(
            _combine_body_g,
            grid=(_GRID,),
            in_specs=in_specs,
            out_specs=[rows, rows],
            out_shape=[jax.ShapeDtypeStruct((_N, _D), jnp.float32)] * 2,
        )(p, h_cur, h0, dinv)
    return pl.pallas_call(
        _combine_body,
        grid=(_GRID,),
        in_specs=in_specs,
        out_specs=rows,
        out_shape=jax.ShapeDtypeStruct((_N, _D), jnp.float32),
    )(p, h_cur, h0, dinv)


def kernel(x, edge_index, W_in, b_in, W_h, b_h, W_out, b_out):
    ei = edge_index.astype(jnp.int32)
    src = ei[0].reshape(_NW, _NB, _B)
    dst = ei[1].reshape(_NW, _NB, _B)
    zeros_d = jnp.zeros((_RPS, _D), jnp.float32)
    ones_b = jnp.ones((_B, _D), jnp.float32)

    h0 = _mlp(x, W_in, b_in.reshape(1, _D), W_h, b_h.reshape(1, _D),
              W_out, b_out.reshape(1, _D))
    degp = _sc_degree(ones_b, dst, zeros_d)
    dinv, g0 = _prep(degp, h0)
    p1 = _sc_scatter(g0, src, dst, zeros_d)
    h1, g1 = _combine(p1, h0, h0, dinv, with_g=True)
    p2 = _sc_scatter(g1, src, dst, zeros_d)
    return _combine(p2, h1, h0, dinv, with_g=False)
